# baseline (speedup 1.0000x reference)
import functools

import jax
import jax.numpy as jnp
from jax import lax
from jax.experimental import pallas as pl
from jax.experimental.pallas import tpu as pltpu
from jax.experimental.pallas import tpu_sc as plsc

_D = 128
_NC = 2
_NS = 16
_NW = _NC * _NS


def _emb_body(rows_per_w, T, group, x_hbm, w_hbm, out_hbm, idx_v, rows_v, g0, g1):
    wid = lax.axis_index("s") * _NC + lax.axis_index("c")
    r0 = wid * rows_per_w
    ngroups = rows_per_w // group

    pltpu.sync_copy(x_hbm.at[pl.ds(r0, rows_per_w)], idx_v)

    sems = (g0, g1)

    def gathers(g, b):
        return [
            pltpu.make_async_copy(
                w_hbm.at[idx_v.at[g * group + j]], rows_v.at[b, j], sems[b])
            for j in range(group)
        ]

    def fire(g, b):
        for c in gathers(g, b):
            c.start()

    def drain(g, b):
        for c in gathers(g, b):
            c.wait()

    def write(g, b):
        pltpu.sync_copy(rows_v.at[b],
                        out_hbm.at[pl.ds(r0 + g * group, group)])

    fire(0, 0)
    fire(1, 1)

    def body(i, carry):
        for b in range(2):
            g = 2 * i + b
            drain(g, b)
            write(g, b)
            fire(g + 2, b)
        return carry

    lax.fori_loop(0, ngroups // 2 - 1, body, 0)

    for b in range(2):
        g = ngroups - 2 + b
        drain(g, b)
        write(g, b)


def kernel(x, weight):
    S, T = x.shape
    rows_per_w = S // _NW
    group = 8
    xi = x.astype(jnp.int32)

    mesh = plsc.VectorSubcoreMesh(core_axis_name="c", subcore_axis_name="s")
    k = pl.kernel(
        functools.partial(_emb_body, rows_per_w, T, group),
        out_type=jax.ShapeDtypeStruct((S, T, _D), jnp.float32),
        mesh=mesh,
        compiler_params=pltpu.CompilerParams(use_tc_tiling_on_sc=True),
        scratch_types=[
            pltpu.VMEM((rows_per_w, T), jnp.int32),
            pltpu.VMEM((2, group, T, _D), jnp.float32),
            pltpu.SemaphoreType.DMA,
            pltpu.SemaphoreType.DMA,
        ],
    )
    return k(xi, weight)
